# baseline (speedup 1.0000x reference)
import functools

import jax
import jax.numpy as jnp
from jax import lax
from jax.experimental import pallas as pl
from jax.experimental.pallas import tpu as pltpu
from jax.experimental.pallas import tpu_sc as plsc


def _sc_gather_t(table_t, ids, n, r):
    info = plsc.get_sparse_core_info()
    nc, ns = info.num_cores, info.num_subcores
    nw = nc * ns
    n_per_w = n // nw
    cpc = 16
    n_chunks = n_per_w // cpc

    mesh = plsc.VectorSubcoreMesh(core_axis_name="c", subcore_axis_name="s")

    @functools.partial(
        pl.kernel,
        mesh=mesh,
        out_type=jax.ShapeDtypeStruct((n, r), jnp.float32),
        scratch_types=[
            pltpu.VMEM((n_per_w,), jnp.int32),
            pltpu.VMEM((r, cpc * 128), jnp.float32),
            pltpu.VMEM((n_per_w, r), jnp.float32),
            pltpu.SemaphoreType.DMA,
            pltpu.SemaphoreType.DMA,
        ],
        compiler_params=pltpu.CompilerParams(needs_layout_passes=False),
    )
    def gather_cols(table_hbm, idx_hbm, out_hbm, idx_v, buf, cols_v, sem, osem):
        wid = lax.axis_index("s") * nc + lax.axis_index("c")
        base = wid * n_per_w
        pltpu.sync_copy(idx_hbm.at[pl.ds(base, n_per_w)], idx_v)
        lanes = jnp.arange(16, dtype=jnp.int32)

        def body(jc, _):
            vec = idx_v[pl.ds(jc * cpc, 16)]
            for k in range(cpc):
                rid = vec[k]
                off = pl.multiple_of((rid // 128) * 128, 128)
                pltpu.async_copy(
                    table_hbm.at[:, pl.ds(off, 128)],
                    buf.at[:, pl.ds(k * 128, 128)],
                    sem,
                )
            pltpu.make_async_copy(
                table_hbm.at[:, pl.ds(0, cpc * 128)], buf, sem
            ).wait()
            for k in range(cpc):
                rid = vec[k]
                col = (rid % 128) + k * 128
                v = plsc.load_gather(buf, [lanes, jnp.broadcast_to(col, (16,))])
                j = jc * cpc + k
                plsc.store_scatter(
                    cols_v, [jnp.broadcast_to(j, (16,)), lanes], v
                )
            return 0

        lax.fori_loop(0, n_chunks, body, 0)
        pltpu.async_copy(cols_v, out_hbm.at[pl.ds(base, n_per_w)], osem).wait()

    return gather_cols(table_t, ids)


def kernel(input_ids, input_states, lora_A, lora_B_w):
    b, s = input_ids.shape
    h = input_states.shape[-1]
    r = lora_A.shape[1]
    n = b * s

    n_parts = 4
    npp = n // n_parts
    blk = 1024
    spc = npp // blk

    ids = input_ids.reshape(n).astype(jnp.int32)
    table_t = lora_A.T
    x2d = input_states.reshape(n, h)
    w_t = lora_B_w.T

    gathered = [
        _sc_gather_t(table_t, lax.slice(ids, (c * npp,), ((c + 1) * npp,)), npp, r)
        for c in range(n_parts)
    ]

    def tc_body(o_alias_ref, g_ref, x_ref, w_ref, o_ref):
        del o_alias_ref
        prj = lax.dot_general(
            g_ref[...],
            w_ref[...],
            dimension_numbers=(((1,), (0,)), ((), ())),
            preferred_element_type=jnp.float32,
        )
        o_ref[...] = x_ref[...] + prj

    out2d = None
    for c in range(n_parts):
        operands = [
            gathered[0] if out2d is None else out2d,
            gathered[c],
            x2d,
            w_t,
        ]
        row0 = c * spc
        out2d = pl.pallas_call(
            tc_body,
            grid=(spc,),
            in_specs=[
                pl.BlockSpec(memory_space=pl.ANY),
                pl.BlockSpec((blk, r), lambda i: (i, 0)),
                pl.BlockSpec((blk, h), lambda i, row0=row0: (row0 + i, 0)),
                pl.BlockSpec((r, h), lambda i: (0, 0)),
            ],
            out_specs=pl.BlockSpec((blk, h), lambda i, row0=row0: (row0 + i, 0)),
            out_shape=jax.ShapeDtypeStruct((n, h), jnp.float32),
            input_output_aliases={} if c == 0 else {0: 0},
        )(*operands)

    return out2d.reshape(b, s, h)


# speedup vs baseline: 1.0307x; 1.0307x over previous
import functools

import jax
import jax.numpy as jnp
from jax import lax
from jax.experimental import pallas as pl
from jax.experimental.pallas import tpu as pltpu
from jax.experimental.pallas import tpu_sc as plsc


def _sc_gather_t(table_t, ids, n, r):
    info = plsc.get_sparse_core_info()
    nc, ns = info.num_cores, info.num_subcores
    nw = nc * ns
    n_per_w = n // nw
    cpc = 32
    n_chunks = n_per_w // cpc

    mesh = plsc.VectorSubcoreMesh(core_axis_name="c", subcore_axis_name="s")

    @functools.partial(
        pl.kernel,
        mesh=mesh,
        out_type=jax.ShapeDtypeStruct((n, r), jnp.float32),
        scratch_types=[
            pltpu.VMEM((n_per_w,), jnp.int32),
            pltpu.VMEM((r, cpc * 128), jnp.float32),
            pltpu.VMEM((n_per_w, r), jnp.float32),
            pltpu.SemaphoreType.DMA,
            pltpu.SemaphoreType.DMA,
        ],
        compiler_params=pltpu.CompilerParams(needs_layout_passes=False),
    )
    def gather_cols(table_hbm, idx_hbm, out_hbm, idx_v, buf, cols_v, sem, osem):
        wid = lax.axis_index("s") * nc + lax.axis_index("c")
        base = wid * n_per_w
        pltpu.sync_copy(idx_hbm.at[pl.ds(base, n_per_w)], idx_v)
        lanes = jnp.arange(16, dtype=jnp.int32)

        def body(jc, _):
            vecs = [
                idx_v[pl.ds(jc * cpc + m * 16, 16)] for m in range(cpc // 16)
            ]
            for k in range(cpc):
                rid = vecs[k // 16][k % 16]
                off = pl.multiple_of((rid // 128) * 128, 128)
                pltpu.async_copy(
                    table_hbm.at[:, pl.ds(off, 128)],
                    buf.at[:, pl.ds(k * 128, 128)],
                    sem,
                )
            pltpu.make_async_copy(
                table_hbm.at[:, pl.ds(0, cpc * 128)], buf, sem
            ).wait()
            for k in range(cpc):
                rid = vecs[k // 16][k % 16]
                col = (rid % 128) + k * 128
                v = plsc.load_gather(buf, [lanes, jnp.broadcast_to(col, (16,))])
                j = jc * cpc + k
                plsc.store_scatter(
                    cols_v, [jnp.broadcast_to(j, (16,)), lanes], v
                )
            return 0

        lax.fori_loop(0, n_chunks, body, 0)
        pltpu.async_copy(cols_v, out_hbm.at[pl.ds(base, n_per_w)], osem).wait()

    return gather_cols(table_t, ids)


def kernel(input_ids, input_states, lora_A, lora_B_w):
    b, s = input_ids.shape
    h = input_states.shape[-1]
    r = lora_A.shape[1]
    n = b * s

    n_parts = 2
    npp = n // n_parts
    blk = 1024
    spc = npp // blk

    ids = input_ids.reshape(n).astype(jnp.int32)
    table_t = lora_A.T
    x2d = input_states.reshape(n, h)
    w_t = lora_B_w.T

    gathered = [
        _sc_gather_t(table_t, lax.slice(ids, (c * npp,), ((c + 1) * npp,)), npp, r)
        for c in range(n_parts)
    ]

    def tc_body(o_alias_ref, g_ref, x_ref, w_ref, o_ref):
        del o_alias_ref
        prj = lax.dot_general(
            g_ref[...],
            w_ref[...],
            dimension_numbers=(((1,), (0,)), ((), ())),
            preferred_element_type=jnp.float32,
        )
        o_ref[...] = x_ref[...] + prj

    out2d = None
    for c in range(n_parts):
        operands = [
            gathered[0] if out2d is None else out2d,
            gathered[c],
            x2d,
            w_t,
        ]
        row0 = c * spc
        out2d = pl.pallas_call(
            tc_body,
            grid=(spc,),
            in_specs=[
                pl.BlockSpec(memory_space=pl.ANY),
                pl.BlockSpec((blk, r), lambda i: (i, 0)),
                pl.BlockSpec((blk, h), lambda i, row0=row0: (row0 + i, 0)),
                pl.BlockSpec((r, h), lambda i: (0, 0)),
            ],
            out_specs=pl.BlockSpec((blk, h), lambda i, row0=row0: (row0 + i, 0)),
            out_shape=jax.ShapeDtypeStruct((n, h), jnp.float32),
            input_output_aliases={} if c == 0 else {0: 0},
        )(*operands)

    return out2d.reshape(b, s, h)
